# Initial kernel scaffold; baseline (speedup 1.0000x reference)
import jax
import jax.numpy as jnp
from jax.experimental import pallas as pl


def kernel(x, edge_index, W0, b0, W1, b1, Wm, bm, Wl, bl):
    raise NotImplementedError("write your pallas kernel here")


# speedup vs baseline: 12.9936x; 12.9936x over previous
import functools

import jax
import jax.numpy as jnp
from jax import lax
from jax.experimental import pallas as pl
from jax.experimental.pallas import tpu as pltpu
from jax.experimental.pallas import tpu_sc as plsc

N_NODES = 10000
N_PAD = 10240
N_EDGES = 320000
IN_CH = 128
HID_CH = 128
OUT_CH = 64

NC = 2
NS = 16
NW = NC * NS
E_PER_TILE = N_EDGES // NW
CHUNK = 80
N_CHUNKS = E_PER_TILE // CHUNK
ROWS_PER_TILE = N_PAD // NS

_MESH = plsc.VectorSubcoreMesh(core_axis_name="c", subcore_axis_name="s")


def _deg_body(dst_hbm, ones_hbm, zeros_hbm, out_hbm, idx_v, ones_v, zrow_v,
              deg_sh):
    c = lax.axis_index("c")
    s = lax.axis_index("s")
    wid = c * NS + s
    pltpu.sync_copy(ones_hbm, ones_v)
    pltpu.sync_copy(zeros_hbm, zrow_v)
    pltpu.sync_copy(zrow_v, deg_sh.at[pl.ds(s * ROWS_PER_TILE, ROWS_PER_TILE)])
    plsc.subcore_barrier()

    base = wid * E_PER_TILE

    def body(i, carry):
        off = pl.multiple_of(base + i * CHUNK, CHUNK)
        pltpu.sync_copy(dst_hbm.at[pl.ds(off, CHUNK)], idx_v)
        pltpu.sync_copy(ones_v, deg_sh.at[idx_v], add=True)
        return carry

    lax.fori_loop(0, N_CHUNKS, body, 0)
    plsc.subcore_barrier()
    pltpu.sync_copy(deg_sh.at[pl.ds(s * ROWS_PER_TILE, ROWS_PER_TILE)], zrow_v)
    pltpu.sync_copy(zrow_v, out_hbm.at[c, pl.ds(s * ROWS_PER_TILE, ROWS_PER_TILE)])


def _deg_partials(dst, ones_c, zeros_r):
    return pl.kernel(
        _deg_body,
        out_type=jax.ShapeDtypeStruct((NC, N_PAD), jnp.float32),
        mesh=_MESH,
        scratch_types=[
            pltpu.VMEM((CHUNK,), jnp.int32),
            pltpu.VMEM((CHUNK,), jnp.float32),
            pltpu.VMEM((ROWS_PER_TILE,), jnp.float32),
            pltpu.VMEM_SHARED((N_PAD,), jnp.float32),
        ],
    )(dst, ones_c, zeros_r)


def _agg_body(y_hbm, src_hbm, dst_hbm, zeros_hbm, out_hbm, srcv, dstv, rows,
              zbuf, sem, agg_sh):
    c = lax.axis_index("c")
    s = lax.axis_index("s")
    wid = c * NS + s
    pltpu.sync_copy(zeros_hbm, zbuf)
    for j in range(5):
        pltpu.sync_copy(zbuf, agg_sh.at[pl.ds(s * ROWS_PER_TILE + j * 128, 128)])
    plsc.subcore_barrier()

    base = wid * E_PER_TILE

    def body(i, carry):
        off = pl.multiple_of(base + i * CHUNK, CHUNK)
        pltpu.sync_copy(src_hbm.at[pl.ds(off, CHUNK)], srcv)
        pltpu.sync_copy(dst_hbm.at[pl.ds(off, CHUNK)], dstv)
        pltpu.async_copy(y_hbm.at[srcv], rows, sem).wait()
        pltpu.sync_copy(rows, agg_sh.at[dstv], add=True)
        return carry

    lax.fori_loop(0, N_CHUNKS, body, 0)
    plsc.subcore_barrier()
    for j in range(5):
        pltpu.sync_copy(agg_sh.at[pl.ds(s * ROWS_PER_TILE + j * 128, 128)], zbuf)
        pltpu.sync_copy(zbuf, out_hbm.at[c, pl.ds(s * ROWS_PER_TILE + j * 128, 128)])


def _agg_partials(y, src, dst, zeros_b):
    return pl.kernel(
        _agg_body,
        out_type=jax.ShapeDtypeStruct((NC, N_PAD, HID_CH), jnp.float32),
        mesh=_MESH,
        scratch_types=[
            pltpu.VMEM((CHUNK,), jnp.int32),
            pltpu.VMEM((CHUNK,), jnp.int32),
            pltpu.VMEM((CHUNK, HID_CH), jnp.float32),
            pltpu.VMEM((128, HID_CH), jnp.float32),
            pltpu.SemaphoreType.DMA,
            pltpu.VMEM_SHARED((N_PAD, HID_CH), jnp.float32),
        ],
    )(y, src, dst, zeros_b)


def _dis(dp_ref):
    deg = dp_ref[:, 0:1] + dp_ref[:, 1:2] + 1.0
    return lax.rsqrt(deg)


def _tc1_body(x_ref, w_ref, dp_ref, y_ref):
    dis = _dis(dp_ref)
    y_ref[...] = jnp.dot(x_ref[...], w_ref[...],
                         preferred_element_type=jnp.float32) * dis


def _tc2_body(p_ref, y0_ref, dp_ref, b_ref, w_ref, y1_ref):
    dis = _dis(dp_ref)
    agg = p_ref[0, :N_NODES, :] + p_ref[1, :N_NODES, :] + y0_ref[...]
    h = jnp.maximum(agg * dis + b_ref[...], 0.0)
    y1_ref[...] = jnp.dot(h, w_ref[...],
                          preferred_element_type=jnp.float32) * dis


def _tc3_body(p_ref, y1_ref, dp_ref, b_ref, wm_ref, bm_ref, wl_ref, bl_ref,
              mean_ref, logstd_ref):
    dis = _dis(dp_ref)
    agg = p_ref[0, :N_NODES, :] + p_ref[1, :N_NODES, :] + y1_ref[...]
    h = jnp.maximum(agg * dis + b_ref[...], 0.0)
    mean_ref[...] = jnp.dot(h, wm_ref[...],
                            preferred_element_type=jnp.float32) + bm_ref[...]
    logstd_ref[...] = jnp.dot(h, wl_ref[...],
                              preferred_element_type=jnp.float32) + bl_ref[...]


_tc1 = pl.pallas_call(
    _tc1_body, out_shape=jax.ShapeDtypeStruct((N_NODES, HID_CH), jnp.float32))
_tc2 = pl.pallas_call(
    _tc2_body, out_shape=jax.ShapeDtypeStruct((N_NODES, HID_CH), jnp.float32))
_tc3 = pl.pallas_call(
    _tc3_body, out_shape=(jax.ShapeDtypeStruct((N_NODES, OUT_CH), jnp.float32),
                          jax.ShapeDtypeStruct((N_NODES, OUT_CH), jnp.float32)))


def kernel(x, edge_index, W0, b0, W1, b1, Wm, bm, Wl, bl):
    src = edge_index[0].astype(jnp.int32)
    dst = edge_index[1].astype(jnp.int32)
    ones_c = jnp.ones((CHUNK,), jnp.float32)
    zeros_r = jnp.zeros((ROWS_PER_TILE,), jnp.float32)
    zeros_b = jnp.zeros((128, HID_CH), jnp.float32)

    dp = _deg_partials(dst, ones_c, zeros_r)
    dpt = dp[:, :N_NODES].T

    y0 = _tc1(x, W0, dpt)
    p0 = _agg_partials(y0, src, dst, zeros_b)
    y1 = _tc2(p0, y0, dpt, b0, W1)
    p1 = _agg_partials(y1, src, dst, zeros_b)
    mean, logstd = _tc3(p1, y1, dpt, b1, Wm, bm, Wl, bl)
    return (mean, logstd)
